# baseline (speedup 1.0000x reference)
import functools
import numpy as np
import jax
import jax.numpy as jnp
from jax import lax
from jax.experimental import pallas as pl
from jax.experimental.pallas import tpu as pltpu
from jax.experimental.pallas import tpu_sc as plsc

_IN_D = 74
_OUT_D = 157
_B = 1024
_S = 256
_NW = 32
_R_PER_W = _B // _NW

_BUCKETS = [
    (14, 20, True, 1500.0),
    (15, 20, True, 1500.0),
    (19, 16, True, 3000.0),
    (56, 16, False, 120.0),
    (57, 16, False, 120.0),
]
_BUCKET_COLS = [c for c, _, _, _ in _BUCKETS]
_PASSTHROUGH = [c for c in range(_IN_D) if c not in _BUCKET_COLS]
_N_PASS = len(_PASSTHROUGH)
_N_OH = _OUT_D - _N_PASS

_PASS_RUNS = []
_rd, _rs, _prev = 0, _PASSTHROUGH[0], _PASSTHROUGH[0] - 1
for _j, _sp in enumerate(_PASSTHROUGH):
    if _sp != _prev + 1:
        _PASS_RUNS.append((_rd, _rs, _j - _rd))
        _rd, _rs = _j, _sp
    _prev = _sp
_PASS_RUNS.append((_rd, _rs, _N_PASS - _rd))

_BLOCK_OFFS = []
_o = 0
for _c, _nb, _sq, _mx in _BUCKETS:
    _BLOCK_OFFS.append(_o)
    _o += _nb


def _sc_body(in_hbm, out_hbm, src0, src1, src2, src3, src4,
             pass_buf0, pass_buf1, pass_buf2, pass_buf3,
             out_buf0, out_buf1, out_buf2, out_buf3,
             src_sem, pass_sem0, pass_sem1, pass_sem2, pass_sem3,
             out_sem0, out_sem1, out_sem2, out_sem3):
    nc = 2
    wid = lax.axis_index("s") * nc + lax.axis_index("c")
    r0 = wid * _R_PER_W
    src_bufs = [src0, src1, src2, src3, src4]
    pass_bufs = [pass_buf0, pass_buf1, pass_buf2, pass_buf3]
    pass_sems = [pass_sem0, pass_sem1, pass_sem2, pass_sem3]
    out_bufs = [out_buf0, out_buf1, out_buf2, out_buf3]
    out_sems = [out_sem0, out_sem1, out_sem2, out_sem3]

    def pass_src_plane(m):
        mi = m.astype(jnp.int32) if hasattr(m, "astype") else jnp.int32(m)
        return (mi + 2 * (mi >= 14).astype(jnp.int32)
                + (mi >= 17).astype(jnp.int32)
                + 2 * (mi >= 53).astype(jnp.int32))

    def pass_in_copy(m, b):
        return pltpu.make_async_copy(
            in_hbm.at[pass_src_plane(m), pl.ds(r0, _R_PER_W), :],
            pass_bufs[b], pass_sems[b])

    def pass_out_copy(m, b):
        return pltpu.make_async_copy(
            pass_bufs[b], out_hbm.at[m, pl.ds(r0, _R_PER_W), :],
            pass_sems[b])

    for i, (src_c, _, _, _) in enumerate(_BUCKETS):
        pltpu.async_copy(in_hbm.at[src_c, pl.ds(r0, _R_PER_W), :],
                         src_bufs[i], src_sem)
    for m in range(4):
        pltpu.async_copy(
            in_hbm.at[pass_src_plane(m), pl.ds(r0, _R_PER_W), :],
            pass_bufs[m], pass_sems[m])
    for i, (src_c, _, _, _) in enumerate(_BUCKETS):
        pltpu.make_async_copy(in_hbm.at[src_c, pl.ds(r0, _R_PER_W), :],
                              src_bufs[i], src_sem).wait()

    def bin_params(k):
        ki = k.astype(jnp.int32)
        slot = ((ki >= _BLOCK_OFFS[1]).astype(jnp.int32)
                + (ki >= _BLOCK_OFFS[2]).astype(jnp.int32)
                + (ki >= _BLOCK_OFFS[3]).astype(jnp.int32)
                + (ki >= _BLOCK_OFFS[4]).astype(jnp.int32))
        t = ki
        for s in range(1, 5):
            t = jnp.where(ki >= _BLOCK_OFFS[s], ki - _BLOCK_OFFS[s], t)
        tf = t.astype(jnp.float32)
        lo = jnp.float32(0.0)
        hi = jnp.float32(0.0)
        for s, (_, nb, is_sqrt, mx) in enumerate(_BUCKETS):
            if is_sqrt:
                sc = np.float32(np.sqrt(mx) / (nb - 1))
                lo_s = (tf * sc) * (tf * sc)
                hi_s = ((tf + 1.0) * sc) * ((tf + 1.0) * sc)
            else:
                sc = np.float32(mx / (nb - 1))
                lo_s = tf * sc
                hi_s = (tf + 1.0) * sc
            hi_s = jnp.where(t == nb - 1, jnp.float32(np.inf), hi_s)
            lo = jnp.where(slot == s, lo_s, lo)
            hi = jnp.where(slot == s, hi_s, hi)
        lo = jnp.where(t == 0, jnp.float32(-np.inf), lo)
        return slot, lo, hi

    def compute_slab(src, lo_vec, hi_vec, dst):
        def row_body(r2, carry):
            for dr in range(2):
                r = r2 * 2 + dr
                for c in range(_S // 16):
                    v = src[r, pl.ds(c * 16, 16)]
                    m = (v >= lo_vec) & (v < hi_vec)
                    dst[r, pl.ds(c * 16, 16)] = jnp.where(m, jnp.float32(1.0),
                                                          jnp.float32(0.0))
            return carry
        lax.fori_loop(0, _R_PER_W // 2, row_body, 0)

    def oh_out_copy(k, b):
        return pltpu.make_async_copy(
            out_bufs[b], out_hbm.at[_N_PASS + k, pl.ds(r0, _R_PER_W), :],
            out_sems[b])

    def step(q, carry):
        for j4 in range(4):
            m = q * 4 + j4
            @pl.when((m >= 4) & (m < _N_PASS))
            def _():
                pass_out_copy(m - 4, j4).wait()
                pltpu.async_copy(
                    in_hbm.at[pass_src_plane(m), pl.ds(r0, _R_PER_W), :],
                    pass_bufs[j4], pass_sems[j4])
            b2 = (j4 + 2) % 4
            @pl.when((m >= 2) & (m - 2 < _N_PASS))
            def _():
                pass_in_copy(m - 2, b2).wait()
                pltpu.async_copy(
                    pass_bufs[b2],
                    out_hbm.at[m - 2, pl.ds(r0, _R_PER_W), :],
                    pass_sems[b2])
            slot, lo, hi = bin_params(m)
            lo_vec = jnp.full((16,), lo, dtype=jnp.float32)
            hi_vec = jnp.full((16,), hi, dtype=jnp.float32)
            @pl.when(m >= 4)
            def _():
                oh_out_copy(m - 4, j4).wait()
            for s in range(5):
                @pl.when(slot == s)
                def _():
                    compute_slab(src_bufs[s], lo_vec, hi_vec, out_bufs[j4])
            pltpu.async_copy(out_bufs[j4],
                             out_hbm.at[_N_PASS + m, pl.ds(r0, _R_PER_W), :],
                             out_sems[j4])
        return carry

    lax.fori_loop(0, _N_OH // 4, step, 0)
    for p in range(_N_PASS - 4, _N_PASS):
        pass_out_copy(p, p % 4).wait()
    for k in range(_N_OH - 4, _N_OH):
        oh_out_copy(k, k % 4).wait()


@jax.jit
def _preprocess(planes):
    mesh = plsc.VectorSubcoreMesh(core_axis_name="c", subcore_axis_name="s")
    k = pl.kernel(
        _sc_body,
        out_type=jax.ShapeDtypeStruct((_OUT_D, _B, _S), jnp.float32),
        mesh=mesh,
        scratch_types=[
            pltpu.VMEM((_R_PER_W, _S), jnp.float32),
            pltpu.VMEM((_R_PER_W, _S), jnp.float32),
            pltpu.VMEM((_R_PER_W, _S), jnp.float32),
            pltpu.VMEM((_R_PER_W, _S), jnp.float32),
            pltpu.VMEM((_R_PER_W, _S), jnp.float32),
            pltpu.VMEM((_R_PER_W, _S), jnp.float32),
            pltpu.VMEM((_R_PER_W, _S), jnp.float32),
            pltpu.VMEM((_R_PER_W, _S), jnp.float32),
            pltpu.VMEM((_R_PER_W, _S), jnp.float32),
            pltpu.VMEM((_R_PER_W, _S), jnp.float32),
            pltpu.VMEM((_R_PER_W, _S), jnp.float32),
            pltpu.VMEM((_R_PER_W, _S), jnp.float32),
            pltpu.VMEM((_R_PER_W, _S), jnp.float32),
            pltpu.SemaphoreType.DMA,
            pltpu.SemaphoreType.DMA,
            pltpu.SemaphoreType.DMA,
            pltpu.SemaphoreType.DMA,
            pltpu.SemaphoreType.DMA,
            pltpu.SemaphoreType.DMA,
            pltpu.SemaphoreType.DMA,
            pltpu.SemaphoreType.DMA,
            pltpu.SemaphoreType.DMA,
        ],
        compiler_params=pltpu.CompilerParams(needs_layout_passes=True),
    )
    return k(planes)


def kernel(features):
    planes = jnp.transpose(features, (2, 0, 1))
    out_planes = _preprocess(planes)
    return jnp.transpose(out_planes, (1, 2, 0))


# speedup vs baseline: 1.2163x; 1.2163x over previous
import functools
import numpy as np
import jax
import jax.numpy as jnp
from jax import lax
from jax.experimental import pallas as pl
from jax.experimental.pallas import tpu as pltpu
from jax.experimental.pallas import tpu_sc as plsc

_IN_D = 74
_OUT_D = 157
_B = 1024
_S = 256
_NW = 32
_R_PER_W = _B // _NW

_BUCKETS = [
    (14, 20, True, 1500.0),
    (15, 20, True, 1500.0),
    (19, 16, True, 3000.0),
    (56, 16, False, 120.0),
    (57, 16, False, 120.0),
]
_BUCKET_COLS = [c for c, _, _, _ in _BUCKETS]
_PASSTHROUGH = [c for c in range(_IN_D) if c not in _BUCKET_COLS]
_N_PASS = len(_PASSTHROUGH)
_N_OH = _OUT_D - _N_PASS

_PASS_RUNS = []
_rd, _rs, _prev = 0, _PASSTHROUGH[0], _PASSTHROUGH[0] - 1
for _j, _sp in enumerate(_PASSTHROUGH):
    if _sp != _prev + 1:
        _PASS_RUNS.append((_rd, _rs, _j - _rd))
        _rd, _rs = _j, _sp
    _prev = _sp
_PASS_RUNS.append((_rd, _rs, _N_PASS - _rd))

_BLOCK_OFFS = []
_o = 0
for _c, _nb, _sq, _mx in _BUCKETS:
    _BLOCK_OFFS.append(_o)
    _o += _nb


def _sc_body(in_hbm, out_hbm, src0, src1, src2, src3, src4,
             pass_buf0, pass_buf1, pass_buf2, pass_buf3,
             out_buf0, out_buf1, out_buf2, out_buf3,
             src_sem, pass_sem0, pass_sem1, pass_sem2, pass_sem3,
             out_sem0, out_sem1, out_sem2, out_sem3):
    nc = 2
    wid = lax.axis_index("s") * nc + lax.axis_index("c")
    r0 = wid * _R_PER_W
    src_bufs = [src0, src1, src2, src3, src4]
    pass_bufs = [pass_buf0, pass_buf1, pass_buf2, pass_buf3]
    pass_sems = [pass_sem0, pass_sem1, pass_sem2, pass_sem3]
    out_bufs = [out_buf0, out_buf1, out_buf2, out_buf3]
    out_sems = [out_sem0, out_sem1, out_sem2, out_sem3]

    def pass_src_plane(m):
        mi = m.astype(jnp.int32) if hasattr(m, "astype") else jnp.int32(m)
        return (mi + 2 * (mi >= 14).astype(jnp.int32)
                + (mi >= 17).astype(jnp.int32)
                + 2 * (mi >= 53).astype(jnp.int32))

    def pass_in_copy(m, b):
        return pltpu.make_async_copy(
            in_hbm.at[pass_src_plane(m), pl.ds(r0, _R_PER_W), :],
            pass_bufs[b], pass_sems[b])

    def pass_out_copy(m, b):
        return pltpu.make_async_copy(
            pass_bufs[b], out_hbm.at[m, pl.ds(r0, _R_PER_W), :],
            pass_sems[b])

    for i, (src_c, _, _, _) in enumerate(_BUCKETS):
        pltpu.async_copy(in_hbm.at[src_c, pl.ds(r0, _R_PER_W), :],
                         src_bufs[i], src_sem)
    for m in range(4):
        pltpu.async_copy(
            in_hbm.at[pass_src_plane(m), pl.ds(r0, _R_PER_W), :],
            pass_bufs[m], pass_sems[m])
    for i, (src_c, _, _, _) in enumerate(_BUCKETS):
        pltpu.make_async_copy(in_hbm.at[src_c, pl.ds(r0, _R_PER_W), :],
                              src_bufs[i], src_sem).wait()

    def bin_params(k):
        ki = k.astype(jnp.int32)
        slot = ((ki >= _BLOCK_OFFS[1]).astype(jnp.int32)
                + (ki >= _BLOCK_OFFS[2]).astype(jnp.int32)
                + (ki >= _BLOCK_OFFS[3]).astype(jnp.int32)
                + (ki >= _BLOCK_OFFS[4]).astype(jnp.int32))
        t = ki
        for s in range(1, 5):
            t = jnp.where(ki >= _BLOCK_OFFS[s], ki - _BLOCK_OFFS[s], t)
        tf = t.astype(jnp.float32)
        lo = jnp.float32(0.0)
        hi = jnp.float32(0.0)
        for s, (_, nb, is_sqrt, mx) in enumerate(_BUCKETS):
            if is_sqrt:
                sc = np.float32(np.sqrt(mx) / (nb - 1))
                lo_s = (tf * sc) * (tf * sc)
                hi_s = ((tf + 1.0) * sc) * ((tf + 1.0) * sc)
            else:
                sc = np.float32(mx / (nb - 1))
                lo_s = tf * sc
                hi_s = (tf + 1.0) * sc
            hi_s = jnp.where(t == nb - 1, jnp.float32(np.inf), hi_s)
            lo = jnp.where(slot == s, lo_s, lo)
            hi = jnp.where(slot == s, hi_s, hi)
        lo = jnp.where(t == 0, jnp.float32(-np.inf), lo)
        return slot, lo, hi

    def compute_slab(src, lo_vec, hi_vec, dst):
        def row_body(r, carry):
            for c in range(_S // 16):
                v = src[r, pl.ds(c * 16, 16)]
                m = (v >= lo_vec) & (v < hi_vec)
                dst[r, pl.ds(c * 16, 16)] = jnp.where(m, jnp.float32(1.0),
                                                      jnp.float32(0.0))
            return carry
        lax.fori_loop(0, _R_PER_W, row_body, 0)

    def oh_out_copy(k, b):
        return pltpu.make_async_copy(
            out_bufs[b], out_hbm.at[_N_PASS + k, pl.ds(r0, _R_PER_W), :],
            out_sems[b])

    def step(q, carry):
        for j4 in range(4):
            m = q * 4 + j4
            @pl.when((m >= 4) & (m < _N_PASS))
            def _():
                pass_out_copy(m - 4, j4).wait()
                pltpu.async_copy(
                    in_hbm.at[pass_src_plane(m), pl.ds(r0, _R_PER_W), :],
                    pass_bufs[j4], pass_sems[j4])
            b2 = (j4 + 2) % 4
            @pl.when((m >= 2) & (m - 2 < _N_PASS))
            def _():
                pass_in_copy(m - 2, b2).wait()
                pltpu.async_copy(
                    pass_bufs[b2],
                    out_hbm.at[m - 2, pl.ds(r0, _R_PER_W), :],
                    pass_sems[b2])
            slot, lo, hi = bin_params(m)
            lo_vec = jnp.full((16,), lo, dtype=jnp.float32)
            hi_vec = jnp.full((16,), hi, dtype=jnp.float32)
            @pl.when(m >= 4)
            def _():
                oh_out_copy(m - 4, j4).wait()
            for s in range(5):
                @pl.when(slot == s)
                def _():
                    compute_slab(src_bufs[s], lo_vec, hi_vec, out_bufs[j4])
            pltpu.async_copy(out_bufs[j4],
                             out_hbm.at[_N_PASS + m, pl.ds(r0, _R_PER_W), :],
                             out_sems[j4])
        return carry

    lax.fori_loop(0, _N_OH // 4, step, 0)
    for p in range(_N_PASS - 4, _N_PASS):
        pass_out_copy(p, p % 4).wait()
    for k in range(_N_OH - 4, _N_OH):
        oh_out_copy(k, k % 4).wait()


@jax.jit
def _preprocess(planes):
    mesh = plsc.VectorSubcoreMesh(core_axis_name="c", subcore_axis_name="s")
    k = pl.kernel(
        _sc_body,
        out_type=jax.ShapeDtypeStruct((_OUT_D, _B, _S), jnp.float32),
        mesh=mesh,
        scratch_types=[
            pltpu.VMEM((_R_PER_W, _S), jnp.float32),
            pltpu.VMEM((_R_PER_W, _S), jnp.float32),
            pltpu.VMEM((_R_PER_W, _S), jnp.float32),
            pltpu.VMEM((_R_PER_W, _S), jnp.float32),
            pltpu.VMEM((_R_PER_W, _S), jnp.float32),
            pltpu.VMEM((_R_PER_W, _S), jnp.float32),
            pltpu.VMEM((_R_PER_W, _S), jnp.float32),
            pltpu.VMEM((_R_PER_W, _S), jnp.float32),
            pltpu.VMEM((_R_PER_W, _S), jnp.float32),
            pltpu.VMEM((_R_PER_W, _S), jnp.float32),
            pltpu.VMEM((_R_PER_W, _S), jnp.float32),
            pltpu.VMEM((_R_PER_W, _S), jnp.float32),
            pltpu.VMEM((_R_PER_W, _S), jnp.float32),
            pltpu.SemaphoreType.DMA,
            pltpu.SemaphoreType.DMA,
            pltpu.SemaphoreType.DMA,
            pltpu.SemaphoreType.DMA,
            pltpu.SemaphoreType.DMA,
            pltpu.SemaphoreType.DMA,
            pltpu.SemaphoreType.DMA,
            pltpu.SemaphoreType.DMA,
            pltpu.SemaphoreType.DMA,
        ],
        compiler_params=pltpu.CompilerParams(needs_layout_passes=True),
    )
    return k(planes)


def kernel(features):
    planes = jnp.transpose(features, (2, 0, 1))
    out_planes = _preprocess(planes)
    return jnp.transpose(out_planes, (1, 2, 0))
